# Initial kernel scaffold; baseline (speedup 1.0000x reference)
import jax
import jax.numpy as jnp
from jax.experimental import pallas as pl


def kernel(rpn_cls_prob, rpn_bbox_pred, anchors, im_info):
    raise NotImplementedError("write your pallas kernel here")


# speedup vs baseline: 17.3907x; 17.3907x over previous
import functools

import jax
import jax.numpy as jnp
from jax import lax
from jax.experimental import pallas as pl
from jax.experimental.pallas import tpu as pltpu
from jax.experimental.pallas import tpu_sc as plsc

N_BOXES = 20000
N_PAD = 20480
N_SUB = 16
CHUNK = N_PAD // N_SUB
NVREG = CHUNK // 16
TOP_N = 300
THR = 0.7
NEG = -3.0e38
BIGF = 3.0e38


def _nms_body(packed_hbm, out_hbm,
              s_sco, s_dx, s_dy, s_dw, s_dh, s_ax1, s_ay1, s_ax2, s_ay2,
              imv, pubv, rbuf, outbuf, shared):
    cid = lax.axis_index("c")
    sid = lax.axis_index("s")
    base = sid * CHUNK

    fields = (s_sco, s_dx, s_dy, s_dw, s_dh, s_ax1, s_ay1, s_ax2, s_ay2)
    for f, ref in enumerate(fields):
        pltpu.sync_copy(packed_hbm.at[pl.ds(f * N_PAD + base, CHUNK)], ref)
    pltpu.sync_copy(packed_hbm.at[pl.ds(9 * N_PAD, 16)], imv)

    lanes_i = lax.iota(jnp.int32, 16)
    lanes_f = lanes_i.astype(jnp.float32)
    im = imv[...]
    im_h = jnp.sum(jnp.where(lanes_i == 0, im, 0.0))
    im_w = jnp.sum(jnp.where(lanes_i == 1, im, 0.0))
    xhi = im_w - 1.0
    yhi = im_h - 1.0

    def decode(i, carry):
        sl = pl.ds(i * 16, 16)
        ax1 = s_ax1[sl]
        ay1 = s_ay1[sl]
        ax2 = s_ax2[sl]
        ay2 = s_ay2[sl]
        dx = s_dx[sl]
        dy = s_dy[sl]
        dw = s_dw[sl]
        dh = s_dh[sl]
        w = ax2 - ax1 + 1.0
        h = ay2 - ay1 + 1.0
        cx = ax1 + 0.5 * w
        cy = ay1 + 0.5 * h
        pcx = dx * w + cx
        pcy = dy * h + cy
        pw = jnp.exp(dw) * w
        ph = jnp.exp(dh) * h
        x1 = pcx - pw * 0.5
        y1 = pcy - ph * 0.5
        x2 = pcx + pw * 0.5
        y2 = pcy + ph * 0.5
        x1 = jnp.minimum(jnp.maximum(x1, 0.0), xhi)
        y1 = jnp.minimum(jnp.maximum(y1, 0.0), yhi)
        x2 = jnp.minimum(jnp.maximum(x2, 0.0), xhi)
        y2 = jnp.minimum(jnp.maximum(y2, 0.0), yhi)
        s_ax1[sl] = x1
        s_ay1[sl] = y1
        s_ax2[sl] = x2
        s_ay2[sl] = y2
        s_dx[sl] = (x2 - x1) * (y2 - y1)
        return carry

    lax.fori_loop(0, NVREG, decode, 0)

    base_f = base.astype(jnp.float32)
    is_writer = jnp.logical_and(cid == 0, sid == 0)

    UNROLL = 4

    def round_body(k, carry):
        wx1, wy1, wx2, wy2, war, widx = carry

        def scan_vreg(i, st):
            curmax, curidx = st
            for j in range(UNROLL):
                sl = pl.ds((i * UNROLL + j) * 16, 16)
                v = s_sco[sl]
                x1 = s_ax1[sl]
                y1 = s_ay1[sl]
                x2 = s_ax2[sl]
                y2 = s_ay2[sl]
                av = s_dx[sl]
                xx1 = jnp.maximum(wx1, x1)
                yy1 = jnp.maximum(wy1, y1)
                xx2 = jnp.minimum(wx2, x2)
                yy2 = jnp.minimum(wy2, y2)
                inter = jnp.maximum(xx2 - xx1, 0.0) * jnp.maximum(yy2 - yy1, 0.0)
                waa = war + av
                sup = jnp.logical_and(inter > THR * (waa - inter), waa > inter)
                gi = ((i * UNROLL + j) * 16 + base).astype(jnp.float32) + lanes_f
                sup = jnp.logical_or(sup, gi == widx)
                v = jnp.where(sup, NEG, v)
                s_sco[sl] = v
                gt = v > curmax
                curidx = jnp.where(gt, gi, curidx)
                curmax = jnp.where(gt, v, curmax)
            return curmax, curidx

        curmax0 = jnp.full((16,), -jnp.inf, jnp.float32)
        curidx0 = jnp.zeros((16,), jnp.float32)
        curmax, curidx = lax.fori_loop(0, NVREG // UNROLL, scan_vreg,
                                       (curmax0, curidx0))

        m = jnp.max(curmax)
        lidx = jnp.min(jnp.where(curmax == m, curidx, BIGF))

        il = jnp.full((16,), (lidx - base_f).astype(jnp.int32))
        cx1 = plsc.load_gather(s_ax1, [il])
        cy1 = plsc.load_gather(s_ay1, [il])
        cx2 = plsc.load_gather(s_ax2, [il])
        cy2 = plsc.load_gather(s_ay2, [il])
        car = plsc.load_gather(s_dx, [il])
        pub = jnp.where(lanes_i == 0, m, 0.0)
        pub = jnp.where(lanes_i == 1, lidx, pub)
        pub = jnp.where(lanes_i == 2, cx1, pub)
        pub = jnp.where(lanes_i == 3, cy1, pub)
        pub = jnp.where(lanes_i == 4, cx2, pub)
        pub = jnp.where(lanes_i == 5, cy2, pub)
        pub = jnp.where(lanes_i == 6, car, pub)
        pubv[...] = pub
        par = lax.rem(k, 2)
        pltpu.sync_copy(pubv, shared.at[par, pl.ds(sid * 16, 16)])
        plsc.subcore_barrier()

        pltpu.sync_copy(shared.at[par], rbuf)

        def splat(v, c):
            idx = jnp.full((16,), c, jnp.int32)
            return v.at[idx].get(mode="promise_in_bounds")

        def board_row(r):
            row = rbuf[pl.ds(r * 16, 16)]
            mine = jnp.full((16,), r, jnp.int32) == sid
            return jnp.where(mine, pub, row)

        best = board_row(0)
        bs = splat(best, 0)
        bi = splat(best, 1)
        for r in range(1, N_SUB):
            row = board_row(r)
            rs = splat(row, 0)
            ri = splat(row, 1)
            take = jnp.logical_or(
                rs > bs, jnp.logical_and(rs == bs, ri < bi))
            best = jnp.where(take, row, best)
            bs = jnp.where(take, rs, bs)
            bi = jnp.where(take, ri, bi)
        nx1 = splat(best, 2)
        ny1 = splat(best, 3)
        nx2 = splat(best, 4)
        ny2 = splat(best, 5)
        nar = splat(best, 6)

        @pl.when(is_writer)
        def _():
            vf = jnp.where(bs > -1.0e37, 1.0, 0.0)
            row = jnp.where(lanes_i == 1, nx1, 0.0)
            row = jnp.where(lanes_i == 2, ny1, row)
            row = jnp.where(lanes_i == 3, nx2, row)
            row = jnp.where(lanes_i == 4, ny2, row)
            row = jnp.where(lanes_i == 5, bs, row)
            outbuf[pl.ds(k * 16, 16)] = row * vf

        return nx1, ny1, nx2, ny2, nar, bi

    init = (jnp.full((16,), 1.0e9, jnp.float32),
            jnp.full((16,), 1.0e9, jnp.float32),
            jnp.full((16,), -1.0e9, jnp.float32),
            jnp.full((16,), -1.0e9, jnp.float32),
            jnp.full((16,), NEG, jnp.float32),
            jnp.full((16,), -1.0, jnp.float32))
    lax.fori_loop(0, TOP_N, round_body, init)

    @pl.when(is_writer)
    def _():
        pltpu.sync_copy(outbuf, out_hbm)


@jax.jit
def _proposal_sc(packed):
    mesh = plsc.VectorSubcoreMesh(core_axis_name="c", subcore_axis_name="s",
                                  num_cores=1)
    f = pl.kernel(
        _nms_body,
        mesh=mesh,
        compiler_params=pltpu.CompilerParams(needs_layout_passes=False),
        out_type=jax.ShapeDtypeStruct((TOP_N * 16,), jnp.float32),
        scratch_types=[
            pltpu.VMEM((CHUNK,), jnp.float32),
            pltpu.VMEM((CHUNK,), jnp.float32),
            pltpu.VMEM((CHUNK,), jnp.float32),
            pltpu.VMEM((CHUNK,), jnp.float32),
            pltpu.VMEM((CHUNK,), jnp.float32),
            pltpu.VMEM((CHUNK,), jnp.float32),
            pltpu.VMEM((CHUNK,), jnp.float32),
            pltpu.VMEM((CHUNK,), jnp.float32),
            pltpu.VMEM((CHUNK,), jnp.float32),
            pltpu.VMEM((16,), jnp.float32),
            pltpu.VMEM((16,), jnp.float32),
            pltpu.VMEM((N_SUB * 16,), jnp.float32),
            pltpu.VMEM((TOP_N * 16,), jnp.float32),
            pltpu.VMEM_SHARED((2, N_SUB * 16), jnp.float32),
        ],
    )
    return f(packed)


def kernel(rpn_cls_prob, rpn_bbox_pred, anchors, im_info):
    scores = jnp.reshape(rpn_cls_prob, (-1, 2))[:, 1]
    deltas = jnp.reshape(rpn_bbox_pred, (-1, 4))
    n = scores.shape[0]
    pad = N_PAD - n
    zpad = jnp.zeros((pad,), jnp.float32)
    packed = jnp.concatenate([
        jnp.concatenate([scores, jnp.full((pad,), NEG, jnp.float32)]),
        jnp.concatenate([deltas[:, 0], zpad]),
        jnp.concatenate([deltas[:, 1], zpad]),
        jnp.concatenate([deltas[:, 2], zpad]),
        jnp.concatenate([deltas[:, 3], zpad]),
        jnp.concatenate([anchors[:, 0], zpad]),
        jnp.concatenate([anchors[:, 1], zpad]),
        jnp.concatenate([anchors[:, 2], zpad]),
        jnp.concatenate([anchors[:, 3], zpad]),
        jnp.pad(im_info.astype(jnp.float32), (0, 16 - im_info.shape[0])),
    ])
    out = _proposal_sc(packed).reshape(TOP_N, 16)
    blob = out[:, 0:5]
    sel_scores = out[:, 5:6]
    return blob, sel_scores


# speedup vs baseline: 19.4246x; 1.1170x over previous
import jax
import jax.numpy as jnp
from jax import lax
from jax.experimental import pallas as pl
from jax.experimental.pallas import tpu as pltpu
from jax.experimental.pallas import tpu_sc as plsc

N_PAD = 20480
N_SUB = 16
CHUNK = N_PAD // N_SUB
NVREG = CHUNK // 16
TOP_N = 300
WPAD = 320
THR = 0.7
NEG = -3.0e38
BIGF = 3.0e38


def _nms_body(packed_hbm, out_hbm,
              s_sco, s_dx, s_dy, s_dw, s_dh, s_ax1, s_ay1, s_ax2, s_ay2,
              imv, pubv, rbuf, outbuf,
              wl_x1, wl_y1, wl_x2, wl_y2, wl_ar,
              shared):
    cid = lax.axis_index("c")
    sid = lax.axis_index("s")
    base = sid * CHUNK

    fields = (s_sco, s_dx, s_dy, s_dw, s_dh, s_ax1, s_ay1, s_ax2, s_ay2)
    for f, ref in enumerate(fields):
        pltpu.sync_copy(packed_hbm.at[pl.ds(f * N_PAD + base, CHUNK)], ref)
    pltpu.sync_copy(packed_hbm.at[pl.ds(9 * N_PAD, 16)], imv)

    lanes_i = lax.iota(jnp.int32, 16)
    lanes_f = lanes_i.astype(jnp.float32)
    im = imv[...]
    im_h = jnp.sum(jnp.where(lanes_i == 0, im, 0.0))
    im_w = jnp.sum(jnp.where(lanes_i == 1, im, 0.0))
    xhi = im_w - 1.0
    yhi = im_h - 1.0

    UNROLL = 4

    def decode(i, carry):
        for j in range(UNROLL):
            sl = pl.ds((i * UNROLL + j) * 16, 16)
            ax1 = s_ax1[sl]
            ay1 = s_ay1[sl]
            ax2 = s_ax2[sl]
            ay2 = s_ay2[sl]
            dx = s_dx[sl]
            dy = s_dy[sl]
            dw = s_dw[sl]
            dh = s_dh[sl]
            w = ax2 - ax1 + 1.0
            h = ay2 - ay1 + 1.0
            cx = ax1 + 0.5 * w
            cy = ay1 + 0.5 * h
            pcx = dx * w + cx
            pcy = dy * h + cy
            pw = jnp.exp(dw) * w
            ph = jnp.exp(dh) * h
            x1 = pcx - pw * 0.5
            y1 = pcy - ph * 0.5
            x2 = pcx + pw * 0.5
            y2 = pcy + ph * 0.5
            x1 = jnp.minimum(jnp.maximum(x1, 0.0), xhi)
            y1 = jnp.minimum(jnp.maximum(y1, 0.0), yhi)
            x2 = jnp.minimum(jnp.maximum(x2, 0.0), xhi)
            y2 = jnp.minimum(jnp.maximum(y2, 0.0), yhi)
            s_ax1[sl] = x1
            s_ay1[sl] = y1
            s_ax2[sl] = x2
            s_ay2[sl] = y2
            s_dx[sl] = (x2 - x1) * (y2 - y1)
        return carry

    lax.fori_loop(0, NVREG // UNROLL, decode, 0)

    def initwl(i, carry):
        sl = pl.ds(i * 16, 16)
        wl_x1[sl] = jnp.full((16,), 1.0e9, jnp.float32)
        wl_y1[sl] = jnp.full((16,), 1.0e9, jnp.float32)
        wl_x2[sl] = jnp.full((16,), -1.0e9, jnp.float32)
        wl_y2[sl] = jnp.full((16,), -1.0e9, jnp.float32)
        wl_ar[sl] = jnp.full((16,), NEG, jnp.float32)
        return carry

    lax.fori_loop(0, WPAD // 16, initwl, 0)

    base_f = base.astype(jnp.float32)
    is_writer = jnp.logical_and(cid == 0, sid == 0)

    def mark_neg(gidx_f):
        loc = jnp.clip((gidx_f - base_f).astype(jnp.int32), 0, CHUNK - 1)
        vq = loc // 16
        ln = loc - vq * 16
        sl = pl.ds(vq * 16, 16)
        v = s_sco[sl]
        s_sco[sl] = jnp.where(lanes_i == ln, NEG, v)

    def argmax_scan():
        def scan_vreg(i, st):
            curmax, curidx = st
            for j in range(UNROLL):
                sl = pl.ds((i * UNROLL + j) * 16, 16)
                v = s_sco[sl]
                gi = ((i * UNROLL + j) * 16 + base).astype(jnp.float32) + lanes_f
                gt = v > curmax
                curidx = jnp.where(gt, gi, curidx)
                curmax = jnp.where(gt, v, curmax)
            return curmax, curidx
        curmax0 = jnp.full((16,), -jnp.inf, jnp.float32)
        curidx0 = jnp.zeros((16,), jnp.float32)
        curmax, curidx = lax.fori_loop(0, NVREG // UNROLL, scan_vreg,
                                       (curmax0, curidx0))
        m = jnp.max(curmax)
        lidx = jnp.min(jnp.where(curmax == m, curidx, BIGF))
        return m, lidx

    def round_body(k, carry):
        nwin16 = carry

        def find_verified(st):
            m, lidx = argmax_scan()
            il = jnp.full((16,), (lidx - base_f).astype(jnp.int32))
            cx1 = plsc.load_gather(s_ax1, [il])
            cy1 = plsc.load_gather(s_ay1, [il])
            cx2 = plsc.load_gather(s_ax2, [il])
            cy2 = plsc.load_gather(s_ay2, [il])
            car = plsc.load_gather(s_dx, [il])

            def vstep(w, rej):
                sl = pl.ds(w * 16, 16)
                wx1 = wl_x1[sl]
                wy1 = wl_y1[sl]
                wx2 = wl_x2[sl]
                wy2 = wl_y2[sl]
                war = wl_ar[sl]
                xx1 = jnp.maximum(wx1, cx1)
                yy1 = jnp.maximum(wy1, cy1)
                xx2 = jnp.minimum(wx2, cx2)
                yy2 = jnp.minimum(wy2, cy2)
                inter = (jnp.maximum(xx2 - xx1, 0.0)
                         * jnp.maximum(yy2 - yy1, 0.0))
                waa = war + car
                sup = jnp.logical_and(inter > THR * (waa - inter),
                                      waa > inter)
                return jnp.logical_or(rej, sup)

            rejv = lax.fori_loop(0, nwin16, vstep,
                                 jnp.zeros((16,), jnp.bool_))
            rejected = jnp.logical_and(jnp.any(rejv), m > -1.0e37)

            @pl.when(rejected)
            def _():
                mark_neg(lidx)
            return m, lidx, cx1, cy1, cx2, cy2, car, rejected

        z16 = jnp.zeros((16,), jnp.float32)
        st0 = (jnp.float32(0), jnp.float32(0), z16, z16, z16, z16, z16,
               jnp.bool_(True))
        m, lidx, cx1, cy1, cx2, cy2, car, _ = lax.while_loop(
            lambda st: st[7], find_verified, st0)

        pub = jnp.where(lanes_i == 0, m, 0.0)
        pub = jnp.where(lanes_i == 1, lidx, pub)
        pub = jnp.where(lanes_i == 2, cx1, pub)
        pub = jnp.where(lanes_i == 3, cy1, pub)
        pub = jnp.where(lanes_i == 4, cx2, pub)
        pub = jnp.where(lanes_i == 5, cy2, pub)
        pub = jnp.where(lanes_i == 6, car, pub)
        pubv[...] = pub
        par = lax.rem(k, 2)
        pltpu.sync_copy(pubv, shared.at[par, pl.ds(sid * 16, 16)])
        plsc.subcore_barrier()
        pltpu.sync_copy(shared.at[par], rbuf)

        def splat(v, c):
            idx = jnp.full((16,), c, jnp.int32)
            return v.at[idx].get(mode="promise_in_bounds")

        def board_row(r):
            row = rbuf[pl.ds(r * 16, 16)]
            mine = jnp.full((16,), r, jnp.int32) == sid
            return jnp.where(mine, pub, row)

        best = board_row(0)
        bs = splat(best, 0)
        bi = splat(best, 1)
        for r in range(1, N_SUB):
            row = board_row(r)
            rs = splat(row, 0)
            ri = splat(row, 1)
            take = jnp.logical_or(
                rs > bs, jnp.logical_and(rs == bs, ri < bi))
            best = jnp.where(take, row, best)
            bs = jnp.where(take, rs, bs)
            bi = jnp.where(take, ri, bi)
        nx1 = splat(best, 2)
        ny1 = splat(best, 3)
        nx2 = splat(best, 4)
        ny2 = splat(best, 5)
        nar = splat(best, 6)
        validv = bs > -1.0e37
        valid = jnp.any(validv)

        widx_s = jnp.max(bi)
        owner0 = jnp.logical_and(
            valid, jnp.logical_and(widx_s >= base_f,
                                   widx_s < base_f + CHUNK))

        @pl.when(owner0)
        def _():
            mark_neg(widx_s)

        @pl.when(valid)
        def _():
            vq = k // 16
            ln = k - vq * 16
            sl = pl.ds(vq * 16, 16)
            sel = lanes_i == ln
            wl_x1[sl] = jnp.where(sel, nx1, wl_x1[sl])
            wl_y1[sl] = jnp.where(sel, ny1, wl_y1[sl])
            wl_x2[sl] = jnp.where(sel, nx2, wl_x2[sl])
            wl_y2[sl] = jnp.where(sel, ny2, wl_y2[sl])
            wl_ar[sl] = jnp.where(sel, nar, wl_ar[sl])

        @pl.when(is_writer)
        def _():
            vf = jnp.where(validv, 1.0, 0.0)
            row = jnp.where(lanes_i == 1, nx1, 0.0)
            row = jnp.where(lanes_i == 2, ny1, row)
            row = jnp.where(lanes_i == 3, nx2, row)
            row = jnp.where(lanes_i == 4, ny2, row)
            row = jnp.where(lanes_i == 5, bs, row)
            outbuf[pl.ds(k * 16, 16)] = row * vf

        return (k + 16) // 16

    lax.fori_loop(0, TOP_N, round_body, jnp.int32(0))

    @pl.when(is_writer)
    def _():
        pltpu.sync_copy(outbuf, out_hbm)


@jax.jit
def _proposal_sc(packed):
    mesh = plsc.VectorSubcoreMesh(core_axis_name="c", subcore_axis_name="s",
                                  num_cores=1)
    f = pl.kernel(
        _nms_body,
        mesh=mesh,
        compiler_params=pltpu.CompilerParams(needs_layout_passes=False),
        out_type=jax.ShapeDtypeStruct((TOP_N * 16,), jnp.float32),
        scratch_types=[
            pltpu.VMEM((CHUNK,), jnp.float32),
            pltpu.VMEM((CHUNK,), jnp.float32),
            pltpu.VMEM((CHUNK,), jnp.float32),
            pltpu.VMEM((CHUNK,), jnp.float32),
            pltpu.VMEM((CHUNK,), jnp.float32),
            pltpu.VMEM((CHUNK,), jnp.float32),
            pltpu.VMEM((CHUNK,), jnp.float32),
            pltpu.VMEM((CHUNK,), jnp.float32),
            pltpu.VMEM((CHUNK,), jnp.float32),
            pltpu.VMEM((16,), jnp.float32),
            pltpu.VMEM((16,), jnp.float32),
            pltpu.VMEM((N_SUB * 16,), jnp.float32),
            pltpu.VMEM((TOP_N * 16,), jnp.float32),
            pltpu.VMEM((WPAD,), jnp.float32),
            pltpu.VMEM((WPAD,), jnp.float32),
            pltpu.VMEM((WPAD,), jnp.float32),
            pltpu.VMEM((WPAD,), jnp.float32),
            pltpu.VMEM((WPAD,), jnp.float32),
            pltpu.VMEM_SHARED((2, N_SUB * 16), jnp.float32),
        ],
    )
    return f(packed)


def kernel(rpn_cls_prob, rpn_bbox_pred, anchors, im_info):
    scores = jnp.reshape(rpn_cls_prob, (-1, 2))[:, 1]
    deltas = jnp.reshape(rpn_bbox_pred, (-1, 4))
    n = scores.shape[0]
    pad = N_PAD - n
    zpad = jnp.zeros((pad,), jnp.float32)
    packed = jnp.concatenate([
        jnp.concatenate([scores, jnp.full((pad,), NEG, jnp.float32)]),
        jnp.concatenate([deltas[:, 0], zpad]),
        jnp.concatenate([deltas[:, 1], zpad]),
        jnp.concatenate([deltas[:, 2], zpad]),
        jnp.concatenate([deltas[:, 3], zpad]),
        jnp.concatenate([anchors[:, 0], zpad]),
        jnp.concatenate([anchors[:, 1], zpad]),
        jnp.concatenate([anchors[:, 2], zpad]),
        jnp.concatenate([anchors[:, 3], zpad]),
        jnp.pad(im_info.astype(jnp.float32), (0, 16 - im_info.shape[0])),
    ])
    out = _proposal_sc(packed).reshape(TOP_N, 16)
    blob = out[:, 0:5]
    sel_scores = out[:, 5:6]
    return blob, sel_scores


# speedup vs baseline: 19.5121x; 1.0045x over previous
import jax
import jax.numpy as jnp
from jax import lax
from jax.experimental import pallas as pl
from jax.experimental.pallas import tpu as pltpu
from jax.experimental.pallas import tpu_sc as plsc

N_PAD = 20480
N_SUB = 16
CHUNK = N_PAD // N_SUB
NVREG = CHUNK // 16
TOP_N = 300
WPAD = 320
THR = 0.7
NEG = -3.0e38
BIGF = 3.0e38


def _nms_body(packed_hbm, out_hbm,
              s_sco, s_dx, s_dy, s_dw, s_dh, s_ax1, s_ay1, s_ax2, s_ay2,
              imv, pubv, rbuf, outbuf,
              wl_x1, wl_y1, wl_x2, wl_y2, wl_ar,
              shared):
    cid = lax.axis_index("c")
    sid = lax.axis_index("s")
    base = sid * CHUNK

    fields = (s_sco, s_dx, s_dy, s_dw, s_dh, s_ax1, s_ay1, s_ax2, s_ay2)
    for f, ref in enumerate(fields):
        pltpu.sync_copy(packed_hbm.at[pl.ds(f * N_PAD + base, CHUNK)], ref)
    pltpu.sync_copy(packed_hbm.at[pl.ds(9 * N_PAD, 16)], imv)

    lanes_i = lax.iota(jnp.int32, 16)
    lanes_f = lanes_i.astype(jnp.float32)
    im = imv[...]
    im_h = jnp.sum(jnp.where(lanes_i == 0, im, 0.0))
    im_w = jnp.sum(jnp.where(lanes_i == 1, im, 0.0))
    xhi = im_w - 1.0
    yhi = im_h - 1.0

    UNROLL = 4

    def decode(i, carry):
        for j in range(UNROLL):
            sl = pl.ds((i * UNROLL + j) * 16, 16)
            ax1 = s_ax1[sl]
            ay1 = s_ay1[sl]
            ax2 = s_ax2[sl]
            ay2 = s_ay2[sl]
            dx = s_dx[sl]
            dy = s_dy[sl]
            dw = s_dw[sl]
            dh = s_dh[sl]
            w = ax2 - ax1 + 1.0
            h = ay2 - ay1 + 1.0
            cx = ax1 + 0.5 * w
            cy = ay1 + 0.5 * h
            pcx = dx * w + cx
            pcy = dy * h + cy
            pw = jnp.exp(dw) * w
            ph = jnp.exp(dh) * h
            x1 = pcx - pw * 0.5
            y1 = pcy - ph * 0.5
            x2 = pcx + pw * 0.5
            y2 = pcy + ph * 0.5
            x1 = jnp.minimum(jnp.maximum(x1, 0.0), xhi)
            y1 = jnp.minimum(jnp.maximum(y1, 0.0), yhi)
            x2 = jnp.minimum(jnp.maximum(x2, 0.0), xhi)
            y2 = jnp.minimum(jnp.maximum(y2, 0.0), yhi)
            s_ax1[sl] = x1
            s_ay1[sl] = y1
            s_ax2[sl] = x2
            s_ay2[sl] = y2
            s_dx[sl] = (x2 - x1) * (y2 - y1)
        return carry

    lax.fori_loop(0, NVREG // UNROLL, decode, 0)

    def initwl(i, carry):
        sl = pl.ds(i * 16, 16)
        wl_x1[sl] = jnp.full((16,), 1.0e9, jnp.float32)
        wl_y1[sl] = jnp.full((16,), 1.0e9, jnp.float32)
        wl_x2[sl] = jnp.full((16,), -1.0e9, jnp.float32)
        wl_y2[sl] = jnp.full((16,), -1.0e9, jnp.float32)
        wl_ar[sl] = jnp.full((16,), NEG, jnp.float32)
        return carry

    lax.fori_loop(0, WPAD // 16, initwl, 0)

    base_f = base.astype(jnp.float32)
    is_writer = jnp.logical_and(cid == 0, sid == 0)

    def mark_neg(gidx_f):
        loc = jnp.clip((gidx_f - base_f).astype(jnp.int32), 0, CHUNK - 1)
        vq = loc // 16
        ln = loc - vq * 16
        sl = pl.ds(vq * 16, 16)
        v = s_sco[sl]
        s_sco[sl] = jnp.where(lanes_i == ln, NEG, v)

    def argmax_scan(excl_idx):
        def scan_vreg(i, st):
            curmax, curidx = st
            for j in range(UNROLL):
                sl = pl.ds((i * UNROLL + j) * 16, 16)
                v = s_sco[sl]
                gi = ((i * UNROLL + j) * 16 + base).astype(jnp.float32) + lanes_f
                v = jnp.where(gi == excl_idx, NEG, v)
                gt = v > curmax
                curidx = jnp.where(gt, gi, curidx)
                curmax = jnp.where(gt, v, curmax)
            return curmax, curidx
        curmax0 = jnp.full((16,), -jnp.inf, jnp.float32)
        curidx0 = jnp.zeros((16,), jnp.float32)
        curmax, curidx = lax.fori_loop(0, NVREG // UNROLL, scan_vreg,
                                       (curmax0, curidx0))
        m = jnp.max(curmax)
        lidx = jnp.min(jnp.where(curmax == m, curidx, BIGF))
        return m, lidx

    def splat(v, c):
        idx = jnp.full((16,), c, jnp.int32)
        return v.at[idx].get(mode="promise_in_bounds")

    def find_verified(nwin16, excl_idx):
        def step(st):
            m, lidx = argmax_scan(excl_idx)
            il = jnp.full((16,), (lidx - base_f).astype(jnp.int32))
            cx1 = plsc.load_gather(s_ax1, [il])
            cy1 = plsc.load_gather(s_ay1, [il])
            cx2 = plsc.load_gather(s_ax2, [il])
            cy2 = plsc.load_gather(s_ay2, [il])
            car = plsc.load_gather(s_dx, [il])

            def vstep(w, rej):
                sl = pl.ds(w * 16, 16)
                wx1 = wl_x1[sl]
                wy1 = wl_y1[sl]
                wx2 = wl_x2[sl]
                wy2 = wl_y2[sl]
                war = wl_ar[sl]
                xx1 = jnp.maximum(wx1, cx1)
                yy1 = jnp.maximum(wy1, cy1)
                xx2 = jnp.minimum(wx2, cx2)
                yy2 = jnp.minimum(wy2, cy2)
                inter = (jnp.maximum(xx2 - xx1, 0.0)
                         * jnp.maximum(yy2 - yy1, 0.0))
                waa = war + car
                sup = jnp.logical_and(inter > THR * (waa - inter),
                                      waa > inter)
                return jnp.logical_or(rej, sup)

            rejv = lax.fori_loop(0, nwin16, vstep,
                                 jnp.zeros((16,), jnp.bool_))
            rejected = jnp.logical_and(jnp.any(rejv), m > -1.0e37)

            @pl.when(rejected)
            def _():
                mark_neg(lidx)
            return m, lidx, cx1, cy1, cx2, cy2, car, rejected

        z16 = jnp.zeros((16,), jnp.float32)
        st0 = (jnp.float32(0), jnp.float32(0), z16, z16, z16, z16, z16,
               jnp.bool_(True))
        m, lidx, cx1, cy1, cx2, cy2, car, _ = lax.while_loop(
            lambda st: st[7], step, st0)
        return m, lidx, cx1, cy1, cx2, cy2, car

    def append_winner(pos, nx1, ny1, nx2, ny2, nar):
        vq = pos // 16
        ln = pos - vq * 16
        sl = pl.ds(vq * 16, 16)
        sel = lanes_i == ln
        wl_x1[sl] = jnp.where(sel, nx1, wl_x1[sl])
        wl_y1[sl] = jnp.where(sel, ny1, wl_y1[sl])
        wl_x2[sl] = jnp.where(sel, nx2, wl_x2[sl])
        wl_y2[sl] = jnp.where(sel, ny2, wl_y2[sl])
        wl_ar[sl] = jnp.where(sel, nar, wl_ar[sl])

    def out_row(pos, x1v, y1v, x2v, y2v, sv, vfv):
        row = jnp.where(lanes_i == 1, x1v, 0.0)
        row = jnp.where(lanes_i == 2, y1v, row)
        row = jnp.where(lanes_i == 3, x2v, row)
        row = jnp.where(lanes_i == 4, y2v, row)
        row = jnp.where(lanes_i == 5, sv, row)
        outbuf[pl.ds(pos * 16, 16)] = row * vfv

    def round_body(st):
        count, rk = st
        nwin16 = (count + 15) // 16

        m1, l1, a_x1, a_y1, a_x2, a_y2, a_ar = find_verified(
            nwin16, jnp.float32(-1.0))
        m2, l2, b_x1, b_y1, b_x2, b_y2, b_ar = find_verified(nwin16, l1)

        pubA = jnp.where(lanes_i == 0, m1, 0.0)
        pubA = jnp.where(lanes_i == 1, l1, pubA)
        pubA = jnp.where(lanes_i == 2, a_x1, pubA)
        pubA = jnp.where(lanes_i == 3, a_y1, pubA)
        pubA = jnp.where(lanes_i == 4, a_x2, pubA)
        pubA = jnp.where(lanes_i == 5, a_y2, pubA)
        pubA = jnp.where(lanes_i == 6, a_ar, pubA)
        pubB = jnp.where(lanes_i == 0, m2, 0.0)
        pubB = jnp.where(lanes_i == 1, l2, pubB)
        pubB = jnp.where(lanes_i == 2, b_x1, pubB)
        pubB = jnp.where(lanes_i == 3, b_y1, pubB)
        pubB = jnp.where(lanes_i == 4, b_x2, pubB)
        pubB = jnp.where(lanes_i == 5, b_y2, pubB)
        pubB = jnp.where(lanes_i == 6, b_ar, pubB)
        pubv[pl.ds(0, 16)] = pubA
        pubv[pl.ds(16, 16)] = pubB
        par = lax.rem(rk, 2)
        pltpu.sync_copy(pubv, shared.at[par, pl.ds(sid * 32, 32)])
        plsc.subcore_barrier()
        pltpu.sync_copy(shared.at[par], rbuf)

        def c1row(r):
            row = rbuf[pl.ds(r * 32, 16)]
            mine = jnp.full((16,), r, jnp.int32) == sid
            return jnp.where(mine, pubA, row)

        def c2row(r):
            row = rbuf[pl.ds(r * 32 + 16, 16)]
            mine = jnp.full((16,), r, jnp.int32) == sid
            return jnp.where(mine, pubB, row)

        best = c1row(0)
        bs = splat(best, 0)
        bi = splat(best, 1)
        for r in range(1, N_SUB):
            row = c1row(r)
            rs = splat(row, 0)
            ri = splat(row, 1)
            take = jnp.logical_or(
                rs > bs, jnp.logical_and(rs == bs, ri < bi))
            best = jnp.where(take, row, best)
            bs = jnp.where(take, rs, bs)
            bi = jnp.where(take, ri, bi)
        wx1 = splat(best, 2)
        wy1 = splat(best, 3)
        wx2 = splat(best, 4)
        wy2 = splat(best, 5)
        war = splat(best, 6)
        validv = bs > -1.0e37
        v1s = jnp.max(bs) > -1.0e37
        w1idx = jnp.max(bi)
        t1 = (w1idx.astype(jnp.int32)) // CHUNK

        @pl.when(jnp.logical_and(
            v1s, jnp.logical_and(w1idx >= base_f,
                                 w1idx < base_f + CHUNK)))
        def _():
            mark_neg(w1idx)

        @pl.when(v1s)
        def _():
            append_winner(count, wx1, wy1, wx2, wy2, war)

        @pl.when(is_writer)
        def _():
            out_row(count, wx1, wy1, wx2, wy2, bs,
                    jnp.where(validv, 1.0, 0.0))

        t1v = jnp.full((16,), t1)
        best2 = None
        k2 = None
        i2 = None
        sup_list = []
        for r in range(N_SUB):
            rmine = jnp.full((16,), r, jnp.int32) == t1v
            cr = jnp.where(rmine, c2row(r), c1row(r))
            sc = splat(cr, 0)
            ci = splat(cr, 1)
            xx1 = jnp.maximum(splat(cr, 2), wx1)
            yy1 = jnp.maximum(splat(cr, 3), wy1)
            xx2 = jnp.minimum(splat(cr, 4), wx2)
            yy2 = jnp.minimum(splat(cr, 5), wy2)
            inter = (jnp.maximum(xx2 - xx1, 0.0)
                     * jnp.maximum(yy2 - yy1, 0.0))
            waa = war + splat(cr, 6)
            supr = jnp.logical_and(inter > THR * (waa - inter),
                                   waa > inter)
            key = jnp.where(supr, NEG, sc)
            bound = splat(c2row(r), 0)
            sup_list.append((supr, bound))
            if best2 is None:
                best2, k2, i2 = cr, key, ci
            else:
                take = jnp.logical_or(
                    key > k2, jnp.logical_and(key == k2, ci < i2))
                best2 = jnp.where(take, cr, best2)
                k2 = jnp.where(take, key, k2)
                i2 = jnp.where(take, ci, i2)
        unsafe = jnp.zeros((16,), jnp.bool_)
        for supr, bound in sup_list:
            unsafe = jnp.logical_or(unsafe,
                                    jnp.logical_and(supr, bound >= k2))
        v2 = jnp.logical_and(jnp.max(k2) > -1.0e37,
                             jnp.logical_and(v1s, count <= TOP_N - 2))
        two = jnp.logical_and(v2, jnp.logical_not(jnp.any(unsafe)))
        w2idx = jnp.max(i2)
        nx1 = splat(best2, 2)
        ny1 = splat(best2, 3)
        nx2 = splat(best2, 4)
        ny2 = splat(best2, 5)
        nar = splat(best2, 6)
        ns = splat(best2, 0)

        @pl.when(jnp.logical_and(
            two, jnp.logical_and(w2idx >= base_f,
                                 w2idx < base_f + CHUNK)))
        def _():
            mark_neg(w2idx)

        @pl.when(two)
        def _():
            append_winner(count + 1, nx1, ny1, nx2, ny2, nar)

        @pl.when(jnp.logical_and(two, is_writer))
        def _():
            out_row(count + 1, nx1, ny1, nx2, ny2, ns,
                    jnp.full((16,), 1.0, jnp.float32))

        return (count + 1 + jnp.where(two, 1, 0).astype(jnp.int32),
                rk + 1)

    lax.while_loop(lambda st: st[0] < TOP_N, round_body,
                   (jnp.int32(0), jnp.int32(0)))

    @pl.when(is_writer)
    def _():
        pltpu.sync_copy(outbuf, out_hbm)


@jax.jit
def _proposal_sc(packed):
    mesh = plsc.VectorSubcoreMesh(core_axis_name="c", subcore_axis_name="s",
                                  num_cores=1)
    f = pl.kernel(
        _nms_body,
        mesh=mesh,
        compiler_params=pltpu.CompilerParams(needs_layout_passes=False),
        out_type=jax.ShapeDtypeStruct((TOP_N * 16,), jnp.float32),
        scratch_types=[
            pltpu.VMEM((CHUNK,), jnp.float32),
            pltpu.VMEM((CHUNK,), jnp.float32),
            pltpu.VMEM((CHUNK,), jnp.float32),
            pltpu.VMEM((CHUNK,), jnp.float32),
            pltpu.VMEM((CHUNK,), jnp.float32),
            pltpu.VMEM((CHUNK,), jnp.float32),
            pltpu.VMEM((CHUNK,), jnp.float32),
            pltpu.VMEM((CHUNK,), jnp.float32),
            pltpu.VMEM((CHUNK,), jnp.float32),
            pltpu.VMEM((16,), jnp.float32),
            pltpu.VMEM((32,), jnp.float32),
            pltpu.VMEM((N_SUB * 32,), jnp.float32),
            pltpu.VMEM((TOP_N * 16,), jnp.float32),
            pltpu.VMEM((WPAD,), jnp.float32),
            pltpu.VMEM((WPAD,), jnp.float32),
            pltpu.VMEM((WPAD,), jnp.float32),
            pltpu.VMEM((WPAD,), jnp.float32),
            pltpu.VMEM((WPAD,), jnp.float32),
            pltpu.VMEM_SHARED((2, N_SUB * 32), jnp.float32),
        ],
    )
    return f(packed)


def kernel(rpn_cls_prob, rpn_bbox_pred, anchors, im_info):
    scores = jnp.reshape(rpn_cls_prob, (-1, 2))[:, 1]
    deltas = jnp.reshape(rpn_bbox_pred, (-1, 4))
    n = scores.shape[0]
    pad = N_PAD - n
    zpad = jnp.zeros((pad,), jnp.float32)
    packed = jnp.concatenate([
        jnp.concatenate([scores, jnp.full((pad,), NEG, jnp.float32)]),
        jnp.concatenate([deltas[:, 0], zpad]),
        jnp.concatenate([deltas[:, 1], zpad]),
        jnp.concatenate([deltas[:, 2], zpad]),
        jnp.concatenate([deltas[:, 3], zpad]),
        jnp.concatenate([anchors[:, 0], zpad]),
        jnp.concatenate([anchors[:, 1], zpad]),
        jnp.concatenate([anchors[:, 2], zpad]),
        jnp.concatenate([anchors[:, 3], zpad]),
        jnp.pad(im_info.astype(jnp.float32), (0, 16 - im_info.shape[0])),
    ])
    out = _proposal_sc(packed).reshape(TOP_N, 16)
    blob = out[:, 0:5]
    sel_scores = out[:, 5:6]
    return blob, sel_scores


# speedup vs baseline: 23.7164x; 1.2155x over previous
import jax
import jax.numpy as jnp
from jax import lax
from jax.experimental import pallas as pl
from jax.experimental.pallas import tpu as pltpu
from jax.experimental.pallas import tpu_sc as plsc

N_PAD = 20480
N_SUB = 16
CHUNK = N_PAD // N_SUB
NVREG = CHUNK // 16
TOP_N = 300
WPAD = 320
THR = 0.7
NEG = -3.0e38
BIGF = 3.0e38


def _nms_body(packed_hbm, out_hbm,
              s_sco, s_dx, s_dy, s_dw, s_dh, s_ax1, s_ay1, s_ax2, s_ay2,
              imv, pubv, rbuf, outbuf,
              wl_x1, wl_y1, wl_x2, wl_y2, wl_ar, s_max,
              shared):
    cid = lax.axis_index("c")
    sid = lax.axis_index("s")
    base = sid * CHUNK

    fields = (s_sco, s_dx, s_dy, s_dw, s_dh, s_ax1, s_ay1, s_ax2, s_ay2)
    for f, ref in enumerate(fields):
        pltpu.sync_copy(packed_hbm.at[pl.ds(f * N_PAD + base, CHUNK)], ref)
    pltpu.sync_copy(packed_hbm.at[pl.ds(9 * N_PAD, 16)], imv)

    lanes_i = lax.iota(jnp.int32, 16)
    lanes_f = lanes_i.astype(jnp.float32)
    im = imv[...]
    im_h = jnp.sum(jnp.where(lanes_i == 0, im, 0.0))
    im_w = jnp.sum(jnp.where(lanes_i == 1, im, 0.0))
    xhi = im_w - 1.0
    yhi = im_h - 1.0

    UNROLL = 4

    def decode(i, carry):
        for j in range(UNROLL):
            sl = pl.ds((i * UNROLL + j) * 16, 16)
            ax1 = s_ax1[sl]
            ay1 = s_ay1[sl]
            ax2 = s_ax2[sl]
            ay2 = s_ay2[sl]
            dx = s_dx[sl]
            dy = s_dy[sl]
            dw = s_dw[sl]
            dh = s_dh[sl]
            w = ax2 - ax1 + 1.0
            h = ay2 - ay1 + 1.0
            cx = ax1 + 0.5 * w
            cy = ay1 + 0.5 * h
            pcx = dx * w + cx
            pcy = dy * h + cy
            pw = jnp.exp(dw) * w
            ph = jnp.exp(dh) * h
            x1 = pcx - pw * 0.5
            y1 = pcy - ph * 0.5
            x2 = pcx + pw * 0.5
            y2 = pcy + ph * 0.5
            x1 = jnp.minimum(jnp.maximum(x1, 0.0), xhi)
            y1 = jnp.minimum(jnp.maximum(y1, 0.0), yhi)
            x2 = jnp.minimum(jnp.maximum(x2, 0.0), xhi)
            y2 = jnp.minimum(jnp.maximum(y2, 0.0), yhi)
            s_ax1[sl] = x1
            s_ay1[sl] = y1
            s_ax2[sl] = x2
            s_ay2[sl] = y2
            s_dx[sl] = (x2 - x1) * (y2 - y1)
        return carry

    lax.fori_loop(0, NVREG // UNROLL, decode, 0)

    def initwl(i, carry):
        sl = pl.ds(i * 16, 16)
        wl_x1[sl] = jnp.full((16,), 1.0e9, jnp.float32)
        wl_y1[sl] = jnp.full((16,), 1.0e9, jnp.float32)
        wl_x2[sl] = jnp.full((16,), -1.0e9, jnp.float32)
        wl_y2[sl] = jnp.full((16,), -1.0e9, jnp.float32)
        wl_ar[sl] = jnp.full((16,), NEG, jnp.float32)
        return carry

    lax.fori_loop(0, WPAD // 16, initwl, 0)

    base_f = base.astype(jnp.float32)
    is_writer = jnp.logical_and(cid == 0, sid == 0)

    NSUM = NVREG // 16

    def build_summary(g, carry):
        row = jnp.full((16,), NEG, jnp.float32)
        for j in range(16):
            v = s_sco[pl.ds((g * 16 + j) * 16, 16)]
            mx = jnp.max(v)
            row = jnp.where(lanes_i == j, mx, row)
        s_max[pl.ds(g * 16, 16)] = row
        return carry

    def mark_neg(gidx_f):
        loc = jnp.clip((gidx_f - base_f).astype(jnp.int32), 0, CHUNK - 1)
        vq = loc // 16
        ln = loc - vq * 16
        sl = pl.ds(vq * 16, 16)
        v2 = jnp.where(lanes_i == ln, NEG, s_sco[sl])
        s_sco[sl] = v2
        mx = jnp.max(v2)
        g = vq // 16
        ln2 = vq - g * 16
        sl2 = pl.ds(g * 16, 16)
        s_max[sl2] = jnp.where(lanes_i == ln2, mx, s_max[sl2])

    def argmax_scan():
        vmax = jnp.full((16,), -jnp.inf, jnp.float32)
        vsid = jnp.zeros((16,), jnp.float32)
        for g in range(NSUM):
            srow = s_max[pl.ds(g * 16, 16)]
            gf = (g * 16 + lanes_i).astype(jnp.float32)
            gt = srow > vmax
            vsid = jnp.where(gt, gf, vsid)
            vmax = jnp.where(gt, srow, vmax)
        m = jnp.max(vmax)
        svq = jnp.min(jnp.where(vmax == m, vsid, BIGF))
        vq = svq.astype(jnp.int32)
        v = s_sco[pl.ds(vq * 16, 16)]
        gi = (vq * 16).astype(jnp.float32) + base_f + lanes_f
        lidx = jnp.min(jnp.where(v == m, gi, BIGF))
        return m, lidx

    def round_body(k, carry):
        nwin16 = carry

        def find_verified(st):
            m, lidx = argmax_scan()
            il = jnp.full((16,), (lidx - base_f).astype(jnp.int32))
            cx1 = plsc.load_gather(s_ax1, [il])
            cy1 = plsc.load_gather(s_ay1, [il])
            cx2 = plsc.load_gather(s_ax2, [il])
            cy2 = plsc.load_gather(s_ay2, [il])
            car = plsc.load_gather(s_dx, [il])

            def vstep(w, rej):
                sl = pl.ds(w * 16, 16)
                wx1 = wl_x1[sl]
                wy1 = wl_y1[sl]
                wx2 = wl_x2[sl]
                wy2 = wl_y2[sl]
                war = wl_ar[sl]
                xx1 = jnp.maximum(wx1, cx1)
                yy1 = jnp.maximum(wy1, cy1)
                xx2 = jnp.minimum(wx2, cx2)
                yy2 = jnp.minimum(wy2, cy2)
                inter = (jnp.maximum(xx2 - xx1, 0.0)
                         * jnp.maximum(yy2 - yy1, 0.0))
                waa = war + car
                sup = jnp.logical_and(inter > THR * (waa - inter),
                                      waa > inter)
                return jnp.logical_or(rej, sup)

            rejv = lax.fori_loop(0, nwin16, vstep,
                                 jnp.zeros((16,), jnp.bool_))
            rejected = jnp.logical_and(jnp.any(rejv), m > -1.0e37)

            @pl.when(rejected)
            def _():
                mark_neg(lidx)
            return m, lidx, cx1, cy1, cx2, cy2, car, rejected

        z16 = jnp.zeros((16,), jnp.float32)
        st0 = (jnp.float32(0), jnp.float32(0), z16, z16, z16, z16, z16,
               jnp.bool_(True))
        m, lidx, cx1, cy1, cx2, cy2, car, _ = lax.while_loop(
            lambda st: st[7], find_verified, st0)

        pub = jnp.where(lanes_i == 0, m, 0.0)
        pub = jnp.where(lanes_i == 1, lidx, pub)
        pub = jnp.where(lanes_i == 2, cx1, pub)
        pub = jnp.where(lanes_i == 3, cy1, pub)
        pub = jnp.where(lanes_i == 4, cx2, pub)
        pub = jnp.where(lanes_i == 5, cy2, pub)
        pub = jnp.where(lanes_i == 6, car, pub)
        pubv[...] = pub
        par = lax.rem(k, 2)
        pltpu.sync_copy(pubv, shared.at[par, pl.ds(sid * 16, 16)])
        plsc.subcore_barrier()
        pltpu.sync_copy(shared.at[par], rbuf)

        def splat(v, c):
            idx = jnp.full((16,), c, jnp.int32)
            return v.at[idx].get(mode="promise_in_bounds")

        def board_row(r):
            row = rbuf[pl.ds(r * 16, 16)]
            mine = jnp.full((16,), r, jnp.int32) == sid
            return jnp.where(mine, pub, row)

        best = board_row(0)
        bs = splat(best, 0)
        bi = splat(best, 1)
        for r in range(1, N_SUB):
            row = board_row(r)
            rs = splat(row, 0)
            ri = splat(row, 1)
            take = jnp.logical_or(
                rs > bs, jnp.logical_and(rs == bs, ri < bi))
            best = jnp.where(take, row, best)
            bs = jnp.where(take, rs, bs)
            bi = jnp.where(take, ri, bi)
        nx1 = splat(best, 2)
        ny1 = splat(best, 3)
        nx2 = splat(best, 4)
        ny2 = splat(best, 5)
        nar = splat(best, 6)
        validv = bs > -1.0e37
        valid = jnp.any(validv)

        widx_s = jnp.max(bi)
        owner0 = jnp.logical_and(
            valid, jnp.logical_and(widx_s >= base_f,
                                   widx_s < base_f + CHUNK))

        @pl.when(owner0)
        def _():
            mark_neg(widx_s)

        @pl.when(valid)
        def _():
            vq = k // 16
            ln = k - vq * 16
            sl = pl.ds(vq * 16, 16)
            sel = lanes_i == ln
            wl_x1[sl] = jnp.where(sel, nx1, wl_x1[sl])
            wl_y1[sl] = jnp.where(sel, ny1, wl_y1[sl])
            wl_x2[sl] = jnp.where(sel, nx2, wl_x2[sl])
            wl_y2[sl] = jnp.where(sel, ny2, wl_y2[sl])
            wl_ar[sl] = jnp.where(sel, nar, wl_ar[sl])

        @pl.when(is_writer)
        def _():
            vf = jnp.where(validv, 1.0, 0.0)
            row = jnp.where(lanes_i == 1, nx1, 0.0)
            row = jnp.where(lanes_i == 2, ny1, row)
            row = jnp.where(lanes_i == 3, nx2, row)
            row = jnp.where(lanes_i == 4, ny2, row)
            row = jnp.where(lanes_i == 5, bs, row)
            outbuf[pl.ds(k * 16, 16)] = row * vf

        return (k + 16) // 16

    lax.fori_loop(0, NSUM, build_summary, 0)
    lax.fori_loop(0, TOP_N, round_body, jnp.int32(0))

    @pl.when(is_writer)
    def _():
        pltpu.sync_copy(outbuf, out_hbm)


@jax.jit
def _proposal_sc(packed):
    mesh = plsc.VectorSubcoreMesh(core_axis_name="c", subcore_axis_name="s",
                                  num_cores=1)
    f = pl.kernel(
        _nms_body,
        mesh=mesh,
        compiler_params=pltpu.CompilerParams(needs_layout_passes=False),
        out_type=jax.ShapeDtypeStruct((TOP_N * 16,), jnp.float32),
        scratch_types=[
            pltpu.VMEM((CHUNK,), jnp.float32),
            pltpu.VMEM((CHUNK,), jnp.float32),
            pltpu.VMEM((CHUNK,), jnp.float32),
            pltpu.VMEM((CHUNK,), jnp.float32),
            pltpu.VMEM((CHUNK,), jnp.float32),
            pltpu.VMEM((CHUNK,), jnp.float32),
            pltpu.VMEM((CHUNK,), jnp.float32),
            pltpu.VMEM((CHUNK,), jnp.float32),
            pltpu.VMEM((CHUNK,), jnp.float32),
            pltpu.VMEM((16,), jnp.float32),
            pltpu.VMEM((16,), jnp.float32),
            pltpu.VMEM((N_SUB * 16,), jnp.float32),
            pltpu.VMEM((TOP_N * 16,), jnp.float32),
            pltpu.VMEM((WPAD,), jnp.float32),
            pltpu.VMEM((WPAD,), jnp.float32),
            pltpu.VMEM((WPAD,), jnp.float32),
            pltpu.VMEM((WPAD,), jnp.float32),
            pltpu.VMEM((WPAD,), jnp.float32),
            pltpu.VMEM((NVREG,), jnp.float32),
            pltpu.VMEM_SHARED((2, N_SUB * 16), jnp.float32),
        ],
    )
    return f(packed)


def kernel(rpn_cls_prob, rpn_bbox_pred, anchors, im_info):
    scores = jnp.reshape(rpn_cls_prob, (-1, 2))[:, 1]
    deltas = jnp.reshape(rpn_bbox_pred, (-1, 4))
    n = scores.shape[0]
    pad = N_PAD - n
    zpad = jnp.zeros((pad,), jnp.float32)
    packed = jnp.concatenate([
        jnp.concatenate([scores, jnp.full((pad,), NEG, jnp.float32)]),
        jnp.concatenate([deltas[:, 0], zpad]),
        jnp.concatenate([deltas[:, 1], zpad]),
        jnp.concatenate([deltas[:, 2], zpad]),
        jnp.concatenate([deltas[:, 3], zpad]),
        jnp.concatenate([anchors[:, 0], zpad]),
        jnp.concatenate([anchors[:, 1], zpad]),
        jnp.concatenate([anchors[:, 2], zpad]),
        jnp.concatenate([anchors[:, 3], zpad]),
        jnp.pad(im_info.astype(jnp.float32), (0, 16 - im_info.shape[0])),
    ])
    out = _proposal_sc(packed).reshape(TOP_N, 16)
    blob = out[:, 0:5]
    sel_scores = out[:, 5:6]
    return blob, sel_scores


# speedup vs baseline: 24.8814x; 1.0491x over previous
import jax
import jax.numpy as jnp
from jax import lax
from jax.experimental import pallas as pl
from jax.experimental.pallas import tpu as pltpu
from jax.experimental.pallas import tpu_sc as plsc

N_PAD = 20480
N_SUB = 16
CHUNK = N_PAD // N_SUB
NVREG = CHUNK // 16
TOP_N = 300
WPAD = 320
THR = 0.7
NEG = -3.0e38
BIGF = 3.0e38


def _nms_body(packed_hbm, out_hbm,
              s_sco, s_dx, s_dy, s_dw, s_dh, s_ax1, s_ay1, s_ax2, s_ay2,
              imv, pubv, rbuf, outbuf,
              wl_x1, wl_y1, wl_x2, wl_y2, wl_ar, s_max,
              shared):
    cid = lax.axis_index("c")
    sid = lax.axis_index("s")
    base = sid * CHUNK

    fields = (s_sco, s_dx, s_dy, s_dw, s_dh, s_ax1, s_ay1, s_ax2, s_ay2)
    for f, ref in enumerate(fields):
        pltpu.sync_copy(packed_hbm.at[pl.ds(f * N_PAD + base, CHUNK)], ref)
    pltpu.sync_copy(packed_hbm.at[pl.ds(9 * N_PAD, 16)], imv)

    lanes_i = lax.iota(jnp.int32, 16)
    lanes_f = lanes_i.astype(jnp.float32)
    im = imv[...]
    im_h = jnp.sum(jnp.where(lanes_i == 0, im, 0.0))
    im_w = jnp.sum(jnp.where(lanes_i == 1, im, 0.0))
    xhi = im_w - 1.0
    yhi = im_h - 1.0

    UNROLL = 4

    def decode(i, carry):
        for j in range(UNROLL):
            sl = pl.ds((i * UNROLL + j) * 16, 16)
            ax1 = s_ax1[sl]
            ay1 = s_ay1[sl]
            ax2 = s_ax2[sl]
            ay2 = s_ay2[sl]
            dx = s_dx[sl]
            dy = s_dy[sl]
            dw = s_dw[sl]
            dh = s_dh[sl]
            w = ax2 - ax1 + 1.0
            h = ay2 - ay1 + 1.0
            cx = ax1 + 0.5 * w
            cy = ay1 + 0.5 * h
            pcx = dx * w + cx
            pcy = dy * h + cy
            pw = jnp.exp(dw) * w
            ph = jnp.exp(dh) * h
            x1 = pcx - pw * 0.5
            y1 = pcy - ph * 0.5
            x2 = pcx + pw * 0.5
            y2 = pcy + ph * 0.5
            x1 = jnp.minimum(jnp.maximum(x1, 0.0), xhi)
            y1 = jnp.minimum(jnp.maximum(y1, 0.0), yhi)
            x2 = jnp.minimum(jnp.maximum(x2, 0.0), xhi)
            y2 = jnp.minimum(jnp.maximum(y2, 0.0), yhi)
            s_ax1[sl] = x1
            s_ay1[sl] = y1
            s_ax2[sl] = x2
            s_ay2[sl] = y2
            s_dx[sl] = (x2 - x1) * (y2 - y1)
        return carry

    lax.fori_loop(0, NVREG // UNROLL, decode, 0)

    def initwl(i, carry):
        sl = pl.ds(i * 16, 16)
        wl_x1[sl] = jnp.full((16,), 1.0e9, jnp.float32)
        wl_y1[sl] = jnp.full((16,), 1.0e9, jnp.float32)
        wl_x2[sl] = jnp.full((16,), -1.0e9, jnp.float32)
        wl_y2[sl] = jnp.full((16,), -1.0e9, jnp.float32)
        wl_ar[sl] = jnp.full((16,), NEG, jnp.float32)
        return carry

    lax.fori_loop(0, WPAD // 16, initwl, 0)

    base_f = base.astype(jnp.float32)
    is_writer = jnp.logical_and(cid == 0, sid == 0)

    NSUM = NVREG // 16

    def build_summary(g, carry):
        row = jnp.full((16,), NEG, jnp.float32)
        for j in range(16):
            v = s_sco[pl.ds((g * 16 + j) * 16, 16)]
            mx = jnp.max(v)
            row = jnp.where(lanes_i == j, mx, row)
        s_max[pl.ds(g * 16, 16)] = row
        return carry

    def set_score(gidx_f, val):
        loc = jnp.clip((gidx_f - base_f).astype(jnp.int32), 0, CHUNK - 1)
        vq = loc // 16
        ln = loc - vq * 16
        sl = pl.ds(vq * 16, 16)
        v2 = jnp.where(lanes_i == ln, val, s_sco[sl])
        s_sco[sl] = v2
        mx = jnp.max(v2)
        g = vq // 16
        ln2 = vq - g * 16
        sl2 = pl.ds(g * 16, 16)
        s_max[sl2] = jnp.where(lanes_i == ln2, mx, s_max[sl2])

    def mark_neg(gidx_f):
        set_score(gidx_f, NEG)

    def argmax_scan():
        vmax = jnp.full((16,), -jnp.inf, jnp.float32)
        vsid = jnp.zeros((16,), jnp.float32)
        for g in range(NSUM):
            srow = s_max[pl.ds(g * 16, 16)]
            gf = (g * 16 + lanes_i).astype(jnp.float32)
            gt = srow > vmax
            vsid = jnp.where(gt, gf, vsid)
            vmax = jnp.where(gt, srow, vmax)
        m = jnp.max(vmax)
        svq = jnp.min(jnp.where(vmax == m, vsid, BIGF))
        vq = svq.astype(jnp.int32)
        v = s_sco[pl.ds(vq * 16, 16)]
        gi = (vq * 16).astype(jnp.float32) + base_f + lanes_f
        lidx = jnp.min(jnp.where(v == m, gi, BIGF))
        return m, lidx

    def splat(v, c):
        idx = jnp.full((16,), c, jnp.int32)
        return v.at[idx].get(mode="promise_in_bounds")

    def find_verified(nwin16):
        def step(st):
            m, lidx = argmax_scan()
            il = jnp.full((16,), (lidx - base_f).astype(jnp.int32))
            cx1 = plsc.load_gather(s_ax1, [il])
            cy1 = plsc.load_gather(s_ay1, [il])
            cx2 = plsc.load_gather(s_ax2, [il])
            cy2 = plsc.load_gather(s_ay2, [il])
            car = plsc.load_gather(s_dx, [il])

            def vstep(w, rej):
                sl = pl.ds(w * 16, 16)
                wx1 = wl_x1[sl]
                wy1 = wl_y1[sl]
                wx2 = wl_x2[sl]
                wy2 = wl_y2[sl]
                war = wl_ar[sl]
                xx1 = jnp.maximum(wx1, cx1)
                yy1 = jnp.maximum(wy1, cy1)
                xx2 = jnp.minimum(wx2, cx2)
                yy2 = jnp.minimum(wy2, cy2)
                inter = (jnp.maximum(xx2 - xx1, 0.0)
                         * jnp.maximum(yy2 - yy1, 0.0))
                waa = war + car
                sup = jnp.logical_and(inter > THR * (waa - inter),
                                      waa > inter)
                return jnp.logical_or(rej, sup)

            rejv = lax.fori_loop(0, nwin16, vstep,
                                 jnp.zeros((16,), jnp.bool_))
            rejected = jnp.logical_and(jnp.any(rejv), m > -1.0e37)

            @pl.when(rejected)
            def _():
                mark_neg(lidx)
            return m, lidx, cx1, cy1, cx2, cy2, car, rejected

        z16 = jnp.zeros((16,), jnp.float32)
        st0 = (jnp.float32(0), jnp.float32(0), z16, z16, z16, z16, z16,
               jnp.bool_(True))
        m, lidx, cx1, cy1, cx2, cy2, car, _ = lax.while_loop(
            lambda st: st[7], step, st0)
        return m, lidx, cx1, cy1, cx2, cy2, car

    def append_winner(pos, nx1, ny1, nx2, ny2, nar):
        vq = pos // 16
        ln = pos - vq * 16
        sl = pl.ds(vq * 16, 16)
        sel = lanes_i == ln
        wl_x1[sl] = jnp.where(sel, nx1, wl_x1[sl])
        wl_y1[sl] = jnp.where(sel, ny1, wl_y1[sl])
        wl_x2[sl] = jnp.where(sel, nx2, wl_x2[sl])
        wl_y2[sl] = jnp.where(sel, ny2, wl_y2[sl])
        wl_ar[sl] = jnp.where(sel, nar, wl_ar[sl])

    def out_row(pos, x1v, y1v, x2v, y2v, sv, vfv):
        row = jnp.where(lanes_i == 1, x1v, 0.0)
        row = jnp.where(lanes_i == 2, y1v, row)
        row = jnp.where(lanes_i == 3, x2v, row)
        row = jnp.where(lanes_i == 4, y2v, row)
        row = jnp.where(lanes_i == 5, sv, row)
        outbuf[pl.ds(pos * 16, 16)] = row * vfv

    def round_body(st):
        count, rk = st
        nwin16 = (count + 15) // 16

        m1, l1, a_x1, a_y1, a_x2, a_y2, a_ar = find_verified(nwin16)
        mark_neg(l1)
        m2, l2, b_x1, b_y1, b_x2, b_y2, b_ar = find_verified(nwin16)
        set_score(l1, m1)

        pubA = jnp.where(lanes_i == 0, m1, 0.0)
        pubA = jnp.where(lanes_i == 1, l1, pubA)
        pubA = jnp.where(lanes_i == 2, a_x1, pubA)
        pubA = jnp.where(lanes_i == 3, a_y1, pubA)
        pubA = jnp.where(lanes_i == 4, a_x2, pubA)
        pubA = jnp.where(lanes_i == 5, a_y2, pubA)
        pubA = jnp.where(lanes_i == 6, a_ar, pubA)
        pubB = jnp.where(lanes_i == 0, m2, 0.0)
        pubB = jnp.where(lanes_i == 1, l2, pubB)
        pubB = jnp.where(lanes_i == 2, b_x1, pubB)
        pubB = jnp.where(lanes_i == 3, b_y1, pubB)
        pubB = jnp.where(lanes_i == 4, b_x2, pubB)
        pubB = jnp.where(lanes_i == 5, b_y2, pubB)
        pubB = jnp.where(lanes_i == 6, b_ar, pubB)
        pubv[pl.ds(0, 16)] = pubA
        pubv[pl.ds(16, 16)] = pubB
        par = lax.rem(rk, 2)
        pltpu.sync_copy(pubv, shared.at[par, pl.ds(sid * 32, 32)])
        plsc.subcore_barrier()
        pltpu.sync_copy(shared.at[par], rbuf)

        def c1row(r):
            row = rbuf[pl.ds(r * 32, 16)]
            mine = jnp.full((16,), r, jnp.int32) == sid
            return jnp.where(mine, pubA, row)

        def c2row(r):
            row = rbuf[pl.ds(r * 32 + 16, 16)]
            mine = jnp.full((16,), r, jnp.int32) == sid
            return jnp.where(mine, pubB, row)

        best = c1row(0)
        bs = splat(best, 0)
        bi = splat(best, 1)
        for r in range(1, N_SUB):
            row = c1row(r)
            rs = splat(row, 0)
            ri = splat(row, 1)
            take = jnp.logical_or(
                rs > bs, jnp.logical_and(rs == bs, ri < bi))
            best = jnp.where(take, row, best)
            bs = jnp.where(take, rs, bs)
            bi = jnp.where(take, ri, bi)
        wx1 = splat(best, 2)
        wy1 = splat(best, 3)
        wx2 = splat(best, 4)
        wy2 = splat(best, 5)
        war = splat(best, 6)
        validv = bs > -1.0e37
        v1s = jnp.max(bs) > -1.0e37
        w1idx = jnp.max(bi)
        t1 = (w1idx.astype(jnp.int32)) // CHUNK

        @pl.when(jnp.logical_and(
            v1s, jnp.logical_and(w1idx >= base_f,
                                 w1idx < base_f + CHUNK)))
        def _():
            mark_neg(w1idx)

        @pl.when(v1s)
        def _():
            append_winner(count, wx1, wy1, wx2, wy2, war)

        @pl.when(is_writer)
        def _():
            out_row(count, wx1, wy1, wx2, wy2, bs,
                    jnp.where(validv, 1.0, 0.0))

        t1v = jnp.full((16,), t1)
        best2 = None
        k2 = None
        i2 = None
        sup_list = []
        for r in range(N_SUB):
            rmine = jnp.full((16,), r, jnp.int32) == t1v
            cr = jnp.where(rmine, c2row(r), c1row(r))
            sc = splat(cr, 0)
            ci = splat(cr, 1)
            xx1 = jnp.maximum(splat(cr, 2), wx1)
            yy1 = jnp.maximum(splat(cr, 3), wy1)
            xx2 = jnp.minimum(splat(cr, 4), wx2)
            yy2 = jnp.minimum(splat(cr, 5), wy2)
            inter = (jnp.maximum(xx2 - xx1, 0.0)
                     * jnp.maximum(yy2 - yy1, 0.0))
            waa = war + splat(cr, 6)
            supr = jnp.logical_and(inter > THR * (waa - inter),
                                   waa > inter)
            key = jnp.where(supr, NEG, sc)
            bound = splat(c2row(r), 0)
            sup_list.append((supr, bound))
            if best2 is None:
                best2, k2, i2 = cr, key, ci
            else:
                take = jnp.logical_or(
                    key > k2, jnp.logical_and(key == k2, ci < i2))
                best2 = jnp.where(take, cr, best2)
                k2 = jnp.where(take, key, k2)
                i2 = jnp.where(take, ci, i2)
        unsafe = jnp.zeros((16,), jnp.bool_)
        for supr, bound in sup_list:
            unsafe = jnp.logical_or(unsafe,
                                    jnp.logical_and(supr, bound >= k2))
        v2 = jnp.logical_and(jnp.max(k2) > -1.0e37,
                             jnp.logical_and(v1s, count <= TOP_N - 2))
        two = jnp.logical_and(v2, jnp.logical_not(jnp.any(unsafe)))
        w2idx = jnp.max(i2)
        nx1 = splat(best2, 2)
        ny1 = splat(best2, 3)
        nx2 = splat(best2, 4)
        ny2 = splat(best2, 5)
        nar = splat(best2, 6)
        ns = splat(best2, 0)

        @pl.when(jnp.logical_and(
            two, jnp.logical_and(w2idx >= base_f,
                                 w2idx < base_f + CHUNK)))
        def _():
            mark_neg(w2idx)

        @pl.when(two)
        def _():
            append_winner(count + 1, nx1, ny1, nx2, ny2, nar)

        @pl.when(jnp.logical_and(two, is_writer))
        def _():
            out_row(count + 1, nx1, ny1, nx2, ny2, ns,
                    jnp.full((16,), 1.0, jnp.float32))

        return (count + 1 + jnp.where(two, 1, 0).astype(jnp.int32),
                rk + 1)

    lax.fori_loop(0, NSUM, build_summary, 0)
    lax.while_loop(lambda st: st[0] < TOP_N, round_body,
                   (jnp.int32(0), jnp.int32(0)))

    @pl.when(is_writer)
    def _():
        pltpu.sync_copy(outbuf, out_hbm)


@jax.jit
def _proposal_sc(packed):
    mesh = plsc.VectorSubcoreMesh(core_axis_name="c", subcore_axis_name="s",
                                  num_cores=1)
    f = pl.kernel(
        _nms_body,
        mesh=mesh,
        compiler_params=pltpu.CompilerParams(needs_layout_passes=False),
        out_type=jax.ShapeDtypeStruct((TOP_N * 16,), jnp.float32),
        scratch_types=[
            pltpu.VMEM((CHUNK,), jnp.float32),
            pltpu.VMEM((CHUNK,), jnp.float32),
            pltpu.VMEM((CHUNK,), jnp.float32),
            pltpu.VMEM((CHUNK,), jnp.float32),
            pltpu.VMEM((CHUNK,), jnp.float32),
            pltpu.VMEM((CHUNK,), jnp.float32),
            pltpu.VMEM((CHUNK,), jnp.float32),
            pltpu.VMEM((CHUNK,), jnp.float32),
            pltpu.VMEM((CHUNK,), jnp.float32),
            pltpu.VMEM((16,), jnp.float32),
            pltpu.VMEM((32,), jnp.float32),
            pltpu.VMEM((N_SUB * 32,), jnp.float32),
            pltpu.VMEM((TOP_N * 16,), jnp.float32),
            pltpu.VMEM((WPAD,), jnp.float32),
            pltpu.VMEM((WPAD,), jnp.float32),
            pltpu.VMEM((WPAD,), jnp.float32),
            pltpu.VMEM((WPAD,), jnp.float32),
            pltpu.VMEM((WPAD,), jnp.float32),
            pltpu.VMEM((NVREG,), jnp.float32),
            pltpu.VMEM_SHARED((2, N_SUB * 32), jnp.float32),
        ],
    )
    return f(packed)


def kernel(rpn_cls_prob, rpn_bbox_pred, anchors, im_info):
    scores = jnp.reshape(rpn_cls_prob, (-1, 2))[:, 1]
    deltas = jnp.reshape(rpn_bbox_pred, (-1, 4))
    n = scores.shape[0]
    pad = N_PAD - n
    zpad = jnp.zeros((pad,), jnp.float32)
    packed = jnp.concatenate([
        jnp.concatenate([scores, jnp.full((pad,), NEG, jnp.float32)]),
        jnp.concatenate([deltas[:, 0], zpad]),
        jnp.concatenate([deltas[:, 1], zpad]),
        jnp.concatenate([deltas[:, 2], zpad]),
        jnp.concatenate([deltas[:, 3], zpad]),
        jnp.concatenate([anchors[:, 0], zpad]),
        jnp.concatenate([anchors[:, 1], zpad]),
        jnp.concatenate([anchors[:, 2], zpad]),
        jnp.concatenate([anchors[:, 3], zpad]),
        jnp.pad(im_info.astype(jnp.float32), (0, 16 - im_info.shape[0])),
    ])
    out = _proposal_sc(packed).reshape(TOP_N, 16)
    blob = out[:, 0:5]
    sel_scores = out[:, 5:6]
    return blob, sel_scores
